# baseline (speedup 1.0000x reference)
import jax
import jax.numpy as jnp
from jax.experimental import pallas as pl
from jax.experimental.pallas import tpu as pltpu

B, H, S, D, Q = 16, 8, 2048, 128, 1
BS = 2048


def _body(pos_ref, c_ref, l_ref, o_ref):
    b = pl.program_id(0)
    s = pl.program_id(1)
    o_ref[...] = c_ref[...]
    local = pos_ref[b] - s * BS

    @pl.when((local >= 0) & (local < BS))
    def _():
        o_ref[0, :, pl.ds(local, 1), :] = l_ref[0]


def _update(cache, latest, pos):
    grid_spec = pltpu.PrefetchScalarGridSpec(
        num_scalar_prefetch=1,
        grid=(B, S // BS),
        in_specs=[
            pl.BlockSpec((1, H, BS, D), lambda b, s, p: (b, 0, s, 0)),
            pl.BlockSpec((1, H, Q, D), lambda b, s, p: (b, 0, 0, 0)),
        ],
        out_specs=pl.BlockSpec((1, H, BS, D), lambda b, s, p: (b, 0, s, 0)),
    )
    return pl.pallas_call(
        _body,
        grid_spec=grid_spec,
        out_shape=jax.ShapeDtypeStruct((B, H, S, D), cache.dtype),
    )(pos, cache, latest)


def kernel(k_cache, v_cache, latest_k, latest_v, position_ids):
    pos = position_ids.reshape(B).astype(jnp.int32)
    k_new = _update(k_cache, latest_k, pos)
    v_new = _update(v_cache, latest_v, pos)
    return (k_new, v_new)


# speedup vs baseline: 1.0029x; 1.0029x over previous
import jax
import jax.numpy as jnp
from jax.experimental import pallas as pl
from jax.experimental.pallas import tpu as pltpu

B, H, S, D, Q = 16, 8, 2048, 128, 1
NBUF = 3


def _body(pos_ref, k_hbm, v_hbm, lk_ref, lv_ref, ok_hbm, ov_hbm,
          buf0, buf1, buf2, isem0, isem1, isem2, osem0, osem1, osem2):
    bufs = (buf0, buf1, buf2)
    isems = (isem0, isem1, isem2)
    osems = (osem0, osem1, osem2)

    slabs = [(k_hbm, ok_hbm, lk_ref, b) for b in range(B)]
    slabs += [(v_hbm, ov_hbm, lv_ref, b) for b in range(B)]
    n = len(slabs)
    prefetch = NBUF - 1

    def start_in(t):
        src, _, _, b = slabs[t]
        cp = pltpu.make_async_copy(src.at[b], bufs[t % NBUF], isems[t % NBUF])
        cp.start()
        return cp

    ins = [None] * NBUF
    outs = [None] * NBUF
    for t in range(prefetch):
        ins[t % NBUF] = start_in(t)
    for t in range(n):
        nb = t % NBUF
        _, dst, lat, b = slabs[t]
        ins[nb].wait()
        local = pos_ref[b]
        bufs[nb][:, pl.ds(local, 1), :] = lat[b].reshape(H, Q, D)
        cp_out = pltpu.make_async_copy(bufs[nb], dst.at[b], osems[nb])
        cp_out.start()
        outs[nb] = cp_out
        tp = t + prefetch
        if tp < n:
            bp = tp % NBUF
            if outs[bp] is not None:
                outs[bp].wait()
                outs[bp] = None
            ins[bp] = start_in(tp)
    for cp in outs:
        if cp is not None:
            cp.wait()


def kernel(k_cache, v_cache, latest_k, latest_v, position_ids):
    pos = position_ids.reshape(B).astype(jnp.int32)
    out_shape = [
        jax.ShapeDtypeStruct((B, H, S, D), k_cache.dtype),
        jax.ShapeDtypeStruct((B, H, S, D), v_cache.dtype),
    ]
    k_new, v_new = pl.pallas_call(
        _body,
        grid=(),
        in_specs=[
            pl.BlockSpec(memory_space=pltpu.SMEM),
            pl.BlockSpec(memory_space=pl.ANY),
            pl.BlockSpec(memory_space=pl.ANY),
            pl.BlockSpec(memory_space=pltpu.VMEM),
            pl.BlockSpec(memory_space=pltpu.VMEM),
        ],
        out_specs=[
            pl.BlockSpec(memory_space=pl.ANY),
            pl.BlockSpec(memory_space=pl.ANY),
        ],
        out_shape=out_shape,
        scratch_shapes=[
            pltpu.VMEM((H, S, D), jnp.float32),
            pltpu.VMEM((H, S, D), jnp.float32),
            pltpu.VMEM((H, S, D), jnp.float32),
            pltpu.SemaphoreType.DMA,
            pltpu.SemaphoreType.DMA,
            pltpu.SemaphoreType.DMA,
            pltpu.SemaphoreType.DMA,
            pltpu.SemaphoreType.DMA,
            pltpu.SemaphoreType.DMA,
        ],
    )(pos, k_cache, v_cache, latest_k, latest_v)
    return (k_new, v_new)


# speedup vs baseline: 1.0057x; 1.0028x over previous
import jax
import jax.numpy as jnp
from jax.experimental import pallas as pl
from jax.experimental.pallas import tpu as pltpu

B, H, S, D, Q = 16, 8, 2048, 128, 1
NBUF = 4


def _body(pos_ref, k_hbm, v_hbm, lk_ref, lv_ref, ok_hbm, ov_hbm,
          buf0, buf1, buf2, buf3, isem0, isem1, isem2, isem3,
          osem0, osem1, osem2, osem3):
    bufs = (buf0, buf1, buf2, buf3)
    isems = (isem0, isem1, isem2, isem3)
    osems = (osem0, osem1, osem2, osem3)

    slabs = [(k_hbm, ok_hbm, lk_ref, b) for b in range(B)]
    slabs += [(v_hbm, ov_hbm, lv_ref, b) for b in range(B)]
    n = len(slabs)
    prefetch = NBUF - 1

    def start_in(t):
        src, _, _, b = slabs[t]
        cp = pltpu.make_async_copy(src.at[b], bufs[t % NBUF], isems[t % NBUF])
        cp.start()
        return cp

    ins = [None] * NBUF
    outs = [None] * NBUF
    for t in range(prefetch):
        ins[t % NBUF] = start_in(t)
    for t in range(n):
        nb = t % NBUF
        _, dst, lat, b = slabs[t]
        ins[nb].wait()
        local = pos_ref[b]
        bufs[nb][:, pl.ds(local, 1), :] = lat[b].reshape(H, Q, D)
        cp_out = pltpu.make_async_copy(bufs[nb], dst.at[b], osems[nb])
        cp_out.start()
        outs[nb] = cp_out
        tp = t + prefetch
        if tp < n:
            bp = tp % NBUF
            if outs[bp] is not None:
                outs[bp].wait()
                outs[bp] = None
            ins[bp] = start_in(tp)
    for cp in outs:
        if cp is not None:
            cp.wait()


def kernel(k_cache, v_cache, latest_k, latest_v, position_ids):
    pos = position_ids.reshape(B).astype(jnp.int32)
    out_shape = [
        jax.ShapeDtypeStruct((B, H, S, D), k_cache.dtype),
        jax.ShapeDtypeStruct((B, H, S, D), v_cache.dtype),
    ]
    k_new, v_new = pl.pallas_call(
        _body,
        grid=(),
        in_specs=[
            pl.BlockSpec(memory_space=pltpu.SMEM),
            pl.BlockSpec(memory_space=pl.ANY),
            pl.BlockSpec(memory_space=pl.ANY),
            pl.BlockSpec(memory_space=pltpu.VMEM),
            pl.BlockSpec(memory_space=pltpu.VMEM),
        ],
        out_specs=[
            pl.BlockSpec(memory_space=pl.ANY),
            pl.BlockSpec(memory_space=pl.ANY),
        ],
        out_shape=out_shape,
        scratch_shapes=[
            pltpu.VMEM((H, S, D), jnp.float32),
            pltpu.VMEM((H, S, D), jnp.float32),
            pltpu.VMEM((H, S, D), jnp.float32),
            pltpu.VMEM((H, S, D), jnp.float32),
            pltpu.SemaphoreType.DMA,
            pltpu.SemaphoreType.DMA,
            pltpu.SemaphoreType.DMA,
            pltpu.SemaphoreType.DMA,
            pltpu.SemaphoreType.DMA,
            pltpu.SemaphoreType.DMA,
            pltpu.SemaphoreType.DMA,
            pltpu.SemaphoreType.DMA,
        ],
    )(pos, k_cache, v_cache, latest_k, latest_v)
    return (k_new, v_new)
